# baseline (speedup 1.0000x reference)
import dataclasses
import functools

import jax
import jax.numpy as jnp
import numpy as np
from jax import lax
from jax.experimental import pallas as pl
from jax.experimental.pallas import tpu as pltpu
from jax.experimental.pallas import tpu_sc as plsc

_N = 10000
_D = 128
_E = 320000

_NC = 2
_NS = 16
_CHUNK = 88
_CPT = 228
_ROWS = _NS * _CPT
_EPAD = _ROWS * _CHUNK
_NH = _N // 2
_NACC = _NH + 256
_XROWS = _N // 2
_XSH = 312
_XLAST = _XROWS - 15 * _XSH
_ASH = 312
_ALAST = _NH - 15 * _ASH
_ZLAST = _NACC - 15 * _ASH

_PERM = np.empty((_D,), dtype=np.int32)
for _g in range(_D // 32):
    for _i in range(16):
        _PERM[32 * _g + 2 * _i] = 32 * _g + _i
        _PERM[32 * _g + 2 * _i + 1] = 32 * _g + 16 + _i


def _sc_body(x_hbm, packed_hbm, zeros_hbm, out_hbm,
             idx0_v, idx1_v, sidx0_v, sidx1_v,
             rows0_v, rows1_v, prod0_v, prod1_v,
             x_shared, acc_shared,
             gsem0, gsem1, ssem0, ssem1, isem0, isem1):
    c = lax.axis_index("c")
    s = lax.axis_index("s")

    idx = (idx0_v, idx1_v)
    sidx = (sidx0_v, sidx1_v)
    rows = (rows0_v, rows1_v)
    prod = (prod0_v, prod1_v)
    gsem = (gsem0, gsem1)
    ssem = (ssem0, ssem1)
    isem = (isem0, isem1)

    @pl.when(s < 15)
    def _():
        pltpu.sync_copy(x_hbm.at[pl.ds(s * _XSH, _XSH)],
                        x_shared.at[pl.ds(s * _XSH, _XSH)])
        pltpu.sync_copy(zeros_hbm.at[pl.ds(s * _ASH, _ASH)],
                        acc_shared.at[pl.ds(s * _ASH, _ASH)])

    @pl.when(s == 15)
    def _():
        pltpu.sync_copy(x_hbm.at[pl.ds(15 * _XSH, _XLAST)],
                        x_shared.at[pl.ds(15 * _XSH, _XLAST)])
        pltpu.sync_copy(zeros_hbm.at[pl.ds(15 * _ASH, _ZLAST)],
                        acc_shared.at[pl.ds(15 * _ASH, _ZLAST)])

    plsc.subcore_barrier()

    base = s * _CPT
    drow = 1 + c

    pltpu.sync_copy(packed_hbm.at[base], idx0_v)
    pltpu.sync_copy(packed_hbm.at[base + 1], idx1_v)
    pltpu.async_copy(x_shared.at[idx0_v.at[0]], rows0_v, gsem0)
    pltpu.async_copy(x_shared.at[idx1_v.at[0]], rows1_v, gsem1)

    three = jnp.broadcast_to(3, (16,)).astype(jnp.int32)
    four = jnp.broadcast_to(4, (16,)).astype(jnp.int32)
    ii = lax.broadcasted_iota(jnp.int32, (16,), 0)
    iig = [ii + 16 * g for g in range(_D // 32)]

    @pl.loop(0, _CPT, step=2)
    def _pair(j):
        for b in range(2):
            ci = j + b
            pltpu.make_async_copy(zeros_hbm.at[pl.ds(0, _CHUNK)],
                                  rows[b], gsem[b]).wait()
            @pl.when(ci >= 2)
            def _():
                pltpu.make_async_copy(zeros_hbm.at[pl.ds(0, _CHUNK)],
                                      prod[b], ssem[b]).wait()

            @plsc.parallel_loop(0, _CHUNK, 1, unroll=4)
            def _edge(e):
                eidx = jnp.broadcast_to(e, (16,)).astype(jnp.int32)
                wv = plsc.bitcast(plsc.load_gather(idx[b], [three, eidx]),
                                  jnp.float32)
                off = plsc.load_gather(idx[b], [four, eidx])
                for g in range(_D // 32):
                    words = plsc.load_gather(rows[b], [eidx, iig[g] + off])
                    ab = plsc.bitcast(words, jnp.bfloat16)
                    lo, hi = plsc.unpack(ab,
                                         format=plsc.PackFormat.INTERLEAVED)
                    prod[b][e, pl.ds(32 * g, 16)] = lo * wv
                    prod[b][e, pl.ds(32 * g + 16, 16)] = hi * wv

            @pl.loop(0, _CHUNK, step=16)
            def _cp(k):
                sidx[b][0, pl.ds(k, 16)] = idx[b][drow, pl.ds(k, 16)]

            pltpu.async_copy(prod[b], acc_shared.at[sidx[b].at[0]],
                             ssem[b], add=True)

            @pl.when(ci + 2 < _CPT)
            def _():
                pltpu.async_copy(packed_hbm.at[base + ci + 2], idx[b],
                                 isem[b])
                pltpu.make_async_copy(packed_hbm.at[base], idx[b],
                                      isem[b]).wait()
                pltpu.async_copy(x_shared.at[idx[b].at[0]], rows[b], gsem[b])

    for b in range(2):
        pltpu.make_async_copy(zeros_hbm.at[pl.ds(0, _CHUNK)],
                              prod[b], ssem[b]).wait()

    plsc.subcore_barrier()
    @pl.when(s < 15)
    def _():
        pltpu.sync_copy(acc_shared.at[pl.ds(s * _ASH, _ASH)],
                        out_hbm.at[pl.ds(c * _NH + s * _ASH, _ASH)])

    @pl.when(s == 15)
    def _():
        pltpu.sync_copy(acc_shared.at[pl.ds(15 * _ASH, _ALAST)],
                        out_hbm.at[pl.ds(c * _NH + 15 * _ASH, _ALAST)])


def _make_sc_call():
    mesh = plsc.VectorSubcoreMesh(core_axis_name="c", subcore_axis_name="s")
    cp = pltpu.CompilerParams()
    if "needs_layout_passes" in pltpu.CompilerParams.__dataclass_fields__:
        cp = dataclasses.replace(cp, needs_layout_passes=False)
    sems = [pltpu.SemaphoreType.DMA] * 6
    return pl.kernel(
        _sc_body,
        out_type=jax.ShapeDtypeStruct((_N, _D), jnp.float32),
        mesh=mesh,
        scratch_types=[
            pltpu.VMEM((5, _CHUNK), jnp.int32),
            pltpu.VMEM((5, _CHUNK), jnp.int32),
            pltpu.VMEM((1, _CHUNK), jnp.int32),
            pltpu.VMEM((1, _CHUNK), jnp.int32),
            pltpu.VMEM((_CHUNK, _D), jnp.int32),
            pltpu.VMEM((_CHUNK, _D), jnp.int32),
            pltpu.VMEM((_CHUNK, _D), jnp.float32),
            pltpu.VMEM((_CHUNK, _D), jnp.float32),
            pltpu.VMEM_SHARED((_XROWS, _D), jnp.int32),
            pltpu.VMEM_SHARED((_NACC, _D), jnp.float32),
        ] + sems,
        compiler_params=cp,
    )


def kernel(x, edge_index, edge_weight):
    src = edge_index[0].astype(jnp.int32)
    dst = edge_index[1].astype(jnp.int32)
    w = edge_weight.astype(jnp.float32)
    pad = _EPAD - _E
    src = jnp.concatenate([src, jnp.zeros((pad,), jnp.int32)]).reshape(_ROWS, _CHUNK)
    dst = jnp.concatenate([dst, jnp.zeros((pad,), jnp.int32)]).reshape(_ROWS, _CHUNK)
    w = jnp.concatenate([w, jnp.zeros((pad,), jnp.float32)]).reshape(_ROWS, _CHUNK)
    wbits = lax.bitcast_convert_type(w, jnp.int32)
    trash = _NH + (jnp.arange(_EPAD, dtype=jnp.int32).reshape(_ROWS, _CHUNK) % 256)
    dloc0 = jnp.where(dst < _NH, dst, trash)
    dloc1 = jnp.where(dst >= _NH, dst - _NH, trash)
    srcrow = src >> 1
    srcoff = (src & 1) * (_D // 2)
    packed = jnp.stack([srcrow, dloc0, dloc1, wbits, srcoff], axis=1)
    xs = lax.bitcast_convert_type(
        x[:, jnp.asarray(_PERM)].astype(jnp.bfloat16).reshape(_N, _D // 2, 2),
        jnp.int32).reshape(_XROWS, _D)
    zeros = jnp.zeros((_NACC, _D), jnp.float32)
    return _make_sc_call()(xs, packed, zeros)


# speedup vs baseline: 1.2071x; 1.2071x over previous
import dataclasses
import functools

import jax
import jax.numpy as jnp
import numpy as np
from jax import lax
from jax.experimental import pallas as pl
from jax.experimental.pallas import tpu as pltpu
from jax.experimental.pallas import tpu_sc as plsc

_N = 10000
_D = 128
_E = 320000

_NC = 2
_NS = 16
_CHUNK = 96
_CPT = 210
_ROWS = _NS * _CPT
_EPAD = _ROWS * _CHUNK
_NH = _N // 2
_NACC = _NH + 8
_XROWS = _N // 2
_XSH = 312
_XLAST = _XROWS - 15 * _XSH
_ASH = 312
_ALAST = _NH - 15 * _ASH
_ZLAST = _NACC - 15 * _ASH

_PERM = np.empty((_D,), dtype=np.int32)
for _g in range(_D // 32):
    for _i in range(16):
        _PERM[32 * _g + 2 * _i] = 32 * _g + _i
        _PERM[32 * _g + 2 * _i + 1] = 32 * _g + 16 + _i


def _sc_body(x_hbm, packed_hbm, zeros_hbm, out_hbm,
             idx0_v, idx1_v, sidx0_v, sidx1_v,
             rows0_v, rows1_v, prod0_v, prod1_v,
             x_shared, acc_shared,
             gsem0, gsem1, ssem0, ssem1, isem0, isem1):
    c = lax.axis_index("c")
    s = lax.axis_index("s")

    idx = (idx0_v, idx1_v)
    sidx = (sidx0_v, sidx1_v)
    rows = (rows0_v, rows1_v)
    prod = (prod0_v, prod1_v)
    gsem = (gsem0, gsem1)
    ssem = (ssem0, ssem1)
    isem = (isem0, isem1)

    @pl.when(s < 15)
    def _():
        pltpu.sync_copy(x_hbm.at[pl.ds(s * _XSH, _XSH)],
                        x_shared.at[pl.ds(s * _XSH, _XSH)])
        pltpu.sync_copy(zeros_hbm.at[pl.ds(s * _ASH, _ASH)],
                        acc_shared.at[pl.ds(s * _ASH, _ASH)])

    @pl.when(s == 15)
    def _():
        pltpu.sync_copy(x_hbm.at[pl.ds(15 * _XSH, _XLAST)],
                        x_shared.at[pl.ds(15 * _XSH, _XLAST)])
        pltpu.sync_copy(zeros_hbm.at[pl.ds(15 * _ASH, _ZLAST)],
                        acc_shared.at[pl.ds(15 * _ASH, _ZLAST)])

    plsc.subcore_barrier()

    base = s * _CPT
    drow = 1 + c

    pltpu.sync_copy(packed_hbm.at[base], idx0_v)
    pltpu.sync_copy(packed_hbm.at[base + 1], idx1_v)
    pltpu.async_copy(x_shared.at[idx0_v.at[0]], rows0_v, gsem0)
    pltpu.async_copy(x_shared.at[idx1_v.at[0]], rows1_v, gsem1)

    three = jnp.broadcast_to(3, (16,)).astype(jnp.int32)
    one = jnp.broadcast_to(1, (16,)).astype(jnp.int32)
    ii = lax.broadcasted_iota(jnp.int32, (16,), 0)
    iig = [ii + 16 * g for g in range(_D // 32)]

    @pl.loop(0, _CPT, step=2)
    def _pair(j):
        for b in range(2):
            ci = j + b
            pltpu.make_async_copy(zeros_hbm.at[pl.ds(0, _CHUNK)],
                                  rows[b], gsem[b]).wait()
            @pl.when(ci >= 2)
            def _():
                pltpu.make_async_copy(zeros_hbm.at[pl.ds(0, _CHUNK)],
                                      prod[b], ssem[b]).wait()

            @plsc.parallel_loop(0, _CHUNK, 1, unroll=4)
            def _edge(e):
                eidx = jnp.broadcast_to(e, (16,)).astype(jnp.int32)
                wraw = plsc.load_gather(idx[b], [three, eidx])
                wv = plsc.bitcast(wraw & ~one, jnp.float32)
                off = (wraw & one) << 6
                for g in range(_D // 32):
                    words = plsc.load_gather(rows[b], [eidx, iig[g] + off])
                    ab = plsc.bitcast(words, jnp.bfloat16)
                    lo, hi = plsc.unpack(ab,
                                         format=plsc.PackFormat.INTERLEAVED)
                    prod[b][e, pl.ds(32 * g, 16)] = lo * wv
                    prod[b][e, pl.ds(32 * g + 16, 16)] = hi * wv

            @pl.loop(0, _CHUNK, step=16)
            def _cp(k):
                sidx[b][0, pl.ds(k, 16)] = idx[b][drow, pl.ds(k, 16)]

            pltpu.async_copy(prod[b], acc_shared.at[sidx[b].at[0]],
                             ssem[b], add=True)

            @pl.when(ci + 2 < _CPT)
            def _():
                pltpu.async_copy(packed_hbm.at[base + ci + 2], idx[b],
                                 isem[b])
                pltpu.make_async_copy(packed_hbm.at[base], idx[b],
                                      isem[b]).wait()
                pltpu.async_copy(x_shared.at[idx[b].at[0]], rows[b], gsem[b])

    for b in range(2):
        pltpu.make_async_copy(zeros_hbm.at[pl.ds(0, _CHUNK)],
                              prod[b], ssem[b]).wait()

    plsc.subcore_barrier()
    @pl.when(s < 15)
    def _():
        pltpu.sync_copy(acc_shared.at[pl.ds(s * _ASH, _ASH)],
                        out_hbm.at[pl.ds(c * _NH + s * _ASH, _ASH)])

    @pl.when(s == 15)
    def _():
        pltpu.sync_copy(acc_shared.at[pl.ds(15 * _ASH, _ALAST)],
                        out_hbm.at[pl.ds(c * _NH + 15 * _ASH, _ALAST)])


def _make_sc_call():
    mesh = plsc.VectorSubcoreMesh(core_axis_name="c", subcore_axis_name="s")
    cp = pltpu.CompilerParams()
    if "needs_layout_passes" in pltpu.CompilerParams.__dataclass_fields__:
        cp = dataclasses.replace(cp, needs_layout_passes=False)
    sems = [pltpu.SemaphoreType.DMA] * 6
    return pl.kernel(
        _sc_body,
        out_type=jax.ShapeDtypeStruct((_N, _D), jnp.float32),
        mesh=mesh,
        scratch_types=[
            pltpu.VMEM((4, _CHUNK), jnp.int32),
            pltpu.VMEM((4, _CHUNK), jnp.int32),
            pltpu.VMEM((1, _CHUNK), jnp.int32),
            pltpu.VMEM((1, _CHUNK), jnp.int32),
            pltpu.VMEM((_CHUNK, _D), jnp.int32),
            pltpu.VMEM((_CHUNK, _D), jnp.int32),
            pltpu.VMEM((_CHUNK, _D), jnp.float32),
            pltpu.VMEM((_CHUNK, _D), jnp.float32),
            pltpu.VMEM_SHARED((_XROWS, _D), jnp.int32),
            pltpu.VMEM_SHARED((_NACC, _D), jnp.float32),
        ] + sems,
        compiler_params=cp,
    )


def kernel(x, edge_index, edge_weight):
    src = edge_index[0].astype(jnp.int32)
    dst = edge_index[1].astype(jnp.int32)
    w = edge_weight.astype(jnp.float32)
    pad = _EPAD - _E
    src = jnp.concatenate([src, jnp.zeros((pad,), jnp.int32)]).reshape(_ROWS, _CHUNK)
    dst = jnp.concatenate([dst, jnp.zeros((pad,), jnp.int32)]).reshape(_ROWS, _CHUNK)
    w = jnp.concatenate([w, jnp.zeros((pad,), jnp.float32)]).reshape(_ROWS, _CHUNK)
    wbits = lax.bitcast_convert_type(w, jnp.int32)
    trash = _NH + (jnp.arange(_EPAD, dtype=jnp.int32).reshape(_ROWS, _CHUNK) % 8)
    dloc0 = jnp.where(dst < _NH, dst, trash)
    dloc1 = jnp.where(dst >= _NH, dst - _NH, trash)
    srcrow = src >> 1
    wbits = (wbits & ~jnp.int32(1)) | (src & 1)
    packed = jnp.stack([srcrow, dloc0, dloc1, wbits], axis=1)
    xs = lax.bitcast_convert_type(
        x[:, jnp.asarray(_PERM)].astype(jnp.bfloat16).reshape(_N, _D // 2, 2),
        jnp.int32).reshape(_XROWS, _D)
    zeros = jnp.zeros((_NACC, _D), jnp.float32)
    return _make_sc_call()(xs, packed, zeros)
